# baseline (speedup 1.0000x reference)
import jax
import jax.numpy as jnp
from jax.experimental import pallas as pl
from jax.experimental.pallas import tpu as pltpu

B, T, D = 16, 2048, 128
H = 256
E = 128
LSTM = 128
NSH = 4
NEG = -1e30

T_BLK = 256
TB = T // T_BLK


def _encoder_kernel(len_ref, state_ref, len2d_ref, w1_ref, b1_ref, w2_ref,
                    b2_ref, wih_ref, whh_ref, bg_ref, out_ref, emb_ref):
    b = pl.program_id(0)
    tb = pl.program_id(1)
    seq_len = len_ref[b]

    @pl.when(tb * T_BLK < seq_len)
    def _embed():
        x = state_ref[0]
        h = jnp.dot(x, w1_ref[:], preferred_element_type=jnp.float32) + b1_ref[:]
        h = jnp.maximum(h, 0.0)
        e = jnp.dot(h, w2_ref[:], preferred_element_type=jnp.float32) + b2_ref[:]
        emb_ref[b, pl.ds(tb * T_BLK, T_BLK), :] = e

    @pl.when(tb * T_BLK >= seq_len)
    def _zero():
        emb_ref[b, pl.ds(tb * T_BLK, T_BLK), :] = jnp.zeros((T_BLK, E),
                                                            jnp.float32)

    @pl.when(jnp.logical_and(b == B - 1, tb == TB - 1))
    def _pool():
        emb = emb_ref[:]
        t_idx = jax.lax.broadcasted_iota(jnp.int32, (B, T), 1)
        valid = t_idx < len2d_ref[:]
        qt = jnp.zeros((B, LSTM), jnp.float32)
        ct = jnp.zeros((B, LSTM), jnp.float32)
        attended = jnp.zeros((B, E), jnp.float32)
        for _ in range(NSH):
            logit = jax.lax.dot_general(
                emb, qt, (((2,), (1,)), ((0,), (0,))),
                preferred_element_type=jnp.float32)
            logit = jnp.where(valid, logit, NEG)
            m = jnp.max(logit, axis=1, keepdims=True)
            w = jnp.exp(logit - m)
            s = jnp.sum(w, axis=1, keepdims=True)
            attended = jax.lax.dot_general(
                w, emb, (((1,), (1,)), ((0,), (0,))),
                preferred_element_type=jnp.float32) / s
            gates = (jnp.dot(attended, wih_ref[:], preferred_element_type=jnp.float32)
                     + jnp.dot(qt, whh_ref[:], preferred_element_type=jnp.float32)
                     + bg_ref[:])
            i_g = jax.nn.sigmoid(gates[:, :LSTM])
            f_g = jax.nn.sigmoid(gates[:, LSTM:2 * LSTM])
            g_g = jnp.tanh(gates[:, 2 * LSTM:3 * LSTM])
            o_g = jax.nn.sigmoid(gates[:, 3 * LSTM:])
            ct = f_g * ct + i_g * g_g
            qt = o_g * jnp.tanh(ct)
        out_ref[:, :E] = attended
        out_ref[:, E:] = qt


def _state_imap(b, tb, len_ref):
    last = jax.lax.div(len_ref[b] + T_BLK - 1, T_BLK) - 1
    return (b, jnp.minimum(tb, last), 0)


def _full(b, tb, len_ref):
    return (0, 0)


def kernel(state, length, W1, b1, W2, b2, W_ih, W_hh, b_ih, b_hh):
    length = length.astype(jnp.int32)
    len2d = length.reshape(B, 1)
    grid_spec = pltpu.PrefetchScalarGridSpec(
        num_scalar_prefetch=1,
        grid=(B, TB),
        in_specs=[
            pl.BlockSpec((1, T_BLK, D), _state_imap),
            pl.BlockSpec((B, 1), _full),
            pl.BlockSpec((D, H), _full),
            pl.BlockSpec((1, H), _full),
            pl.BlockSpec((H, E), _full),
            pl.BlockSpec((1, E), _full),
            pl.BlockSpec((E, 4 * LSTM), _full),
            pl.BlockSpec((LSTM, 4 * LSTM), _full),
            pl.BlockSpec((1, 4 * LSTM), _full),
        ],
        out_specs=pl.BlockSpec((B, E + LSTM), _full),
        scratch_shapes=[pltpu.VMEM((B, T, E), jnp.float32)],
    )
    return pl.pallas_call(
        _encoder_kernel,
        grid_spec=grid_spec,
        out_shape=jax.ShapeDtypeStruct((B, E + LSTM), jnp.float32),
        compiler_params=pltpu.CompilerParams(
            dimension_semantics=("arbitrary", "arbitrary")),
    )(length, state, len2d, W1.T, b1.reshape(1, H), W2.T, b2.reshape(1, E),
      W_ih.T, W_hh.T, (b_ih + b_hh).reshape(1, 4 * LSTM))


# speedup vs baseline: 2.0600x; 2.0600x over previous
import jax
import jax.numpy as jnp
from jax.experimental import pallas as pl
from jax.experimental.pallas import tpu as pltpu

B, T, D = 16, 2048, 128
H = 256
E = 128
LSTM = 128
NSH = 4
NEG = -1e30

T_BLK = 1024
TB = T // T_BLK


def _encoder_kernel(len_ref, state_ref, len2d_ref, w1_ref, b1_ref, w2_ref,
                    b2_ref, wih_ref, whh_ref, bg_ref, out_ref, emb_ref):
    b = pl.program_id(0)
    tb = pl.program_id(1)
    seq_len = len_ref[b]
    row0 = b * T + tb * T_BLK

    @pl.when(tb * T_BLK < seq_len)
    def _embed():
        x = state_ref[0]
        h = jnp.dot(x, w1_ref[:], preferred_element_type=jnp.float32) + b1_ref[:]
        h = jnp.maximum(h, 0.0)
        e = jnp.dot(h, w2_ref[:], preferred_element_type=jnp.float32) + b2_ref[:]
        emb_ref[pl.ds(row0, T_BLK), :] = e

    @pl.when(tb * T_BLK >= seq_len)
    def _zero():
        emb_ref[pl.ds(row0, T_BLK), :] = jnp.zeros((T_BLK, E), jnp.float32)

    @pl.when(jnp.logical_and(b == B - 1, tb == TB - 1))
    def _pool():
        t_idx = jax.lax.broadcasted_iota(jnp.int32, (B, T), 1)
        valid = t_idx < len2d_ref[:]
        qt = jnp.zeros((B, LSTM), jnp.float32)
        ct = jnp.zeros((B, LSTM), jnp.float32)
        attended = jnp.zeros((B, E), jnp.float32)
        for _ in range(NSH):
            rows = []
            for bb in range(B):
                sl = emb_ref[bb * T:(bb + 1) * T, :]
                rows.append(jax.lax.dot_general(
                    qt[bb:bb + 1, :], sl, (((1,), (1,)), ((), ())),
                    preferred_element_type=jnp.float32))
            logit = jnp.concatenate(rows, axis=0)
            logit = jnp.where(valid, logit, NEG)
            m = jnp.max(logit, axis=1, keepdims=True)
            w = jnp.exp(logit - m)
            s = jnp.sum(w, axis=1, keepdims=True)
            rows = []
            for bb in range(B):
                sl = emb_ref[bb * T:(bb + 1) * T, :]
                rows.append(jax.lax.dot_general(
                    w[bb:bb + 1, :], sl, (((1,), (0,)), ((), ())),
                    preferred_element_type=jnp.float32))
            attended = jnp.concatenate(rows, axis=0) / s
            gates = (jnp.dot(attended, wih_ref[:], preferred_element_type=jnp.float32)
                     + jnp.dot(qt, whh_ref[:], preferred_element_type=jnp.float32)
                     + bg_ref[:])
            i_g = jax.nn.sigmoid(gates[:, :LSTM])
            f_g = jax.nn.sigmoid(gates[:, LSTM:2 * LSTM])
            g_g = jnp.tanh(gates[:, 2 * LSTM:3 * LSTM])
            o_g = jax.nn.sigmoid(gates[:, 3 * LSTM:])
            ct = f_g * ct + i_g * g_g
            qt = o_g * jnp.tanh(ct)
        out_ref[:, :E] = attended
        out_ref[:, E:] = qt


def _state_imap(b, tb, len_ref):
    last = jax.lax.div(len_ref[b] + T_BLK - 1, T_BLK) - 1
    return (b, jnp.minimum(tb, last), 0)


def _full(b, tb, len_ref):
    return (0, 0)


def kernel(state, length, W1, b1, W2, b2, W_ih, W_hh, b_ih, b_hh):
    length = length.astype(jnp.int32)
    len2d = length.reshape(B, 1)
    grid_spec = pltpu.PrefetchScalarGridSpec(
        num_scalar_prefetch=1,
        grid=(B, TB),
        in_specs=[
            pl.BlockSpec((1, T_BLK, D), _state_imap),
            pl.BlockSpec((B, 1), _full),
            pl.BlockSpec((D, H), _full),
            pl.BlockSpec((1, H), _full),
            pl.BlockSpec((H, E), _full),
            pl.BlockSpec((1, E), _full),
            pl.BlockSpec((E, 4 * LSTM), _full),
            pl.BlockSpec((LSTM, 4 * LSTM), _full),
            pl.BlockSpec((1, 4 * LSTM), _full),
        ],
        out_specs=pl.BlockSpec((B, E + LSTM), _full),
        scratch_shapes=[pltpu.VMEM((B * T, E), jnp.float32)],
    )
    return pl.pallas_call(
        _encoder_kernel,
        grid_spec=grid_spec,
        out_shape=jax.ShapeDtypeStruct((B, E + LSTM), jnp.float32),
        compiler_params=pltpu.CompilerParams(
            dimension_semantics=("arbitrary", "arbitrary")),
    )(length, state, len2d, W1.T, b1.reshape(1, H), W2.T, b2.reshape(1, E),
      W_ih.T, W_hh.T, (b_ih + b_hh).reshape(1, 4 * LSTM))


# speedup vs baseline: 2.0616x; 1.0008x over previous
import jax
import jax.numpy as jnp
from jax.experimental import pallas as pl
from jax.experimental.pallas import tpu as pltpu

B, T, D = 16, 2048, 128
H = 256
E = 128
LSTM = 128
NSH = 4
NEG = -1e30

T_BLK = 1024
TB = T // T_BLK


def _encoder_kernel(len_ref, state_ref, len2d_ref, w1_ref, b1_ref, w2_ref,
                    b2_ref, wih_ref, whh_ref, bg_ref, out_ref, emb_ref):
    b = pl.program_id(0)
    tb = pl.program_id(1)
    seq_len = len_ref[b]
    row0 = b * T + tb * T_BLK

    @pl.when(tb * T_BLK < seq_len)
    def _embed():
        x = state_ref[0].astype(jnp.bfloat16)
        h = jnp.dot(x, w1_ref[:], preferred_element_type=jnp.float32) + b1_ref[:]
        h = jnp.maximum(h, 0.0).astype(jnp.bfloat16)
        e = jnp.dot(h, w2_ref[:], preferred_element_type=jnp.float32) + b2_ref[:]
        emb_ref[pl.ds(row0, T_BLK), :] = e

    @pl.when(tb * T_BLK >= seq_len)
    def _zero():
        emb_ref[pl.ds(row0, T_BLK), :] = jnp.zeros((T_BLK, E), jnp.float32)

    @pl.when(jnp.logical_and(b == B - 1, tb == TB - 1))
    def _pool():
        t_idx = jax.lax.broadcasted_iota(jnp.int32, (B, T), 1)
        valid = t_idx < len2d_ref[:]
        qt = jnp.zeros((B, LSTM), jnp.float32)
        ct = jnp.zeros((B, LSTM), jnp.float32)
        attended = jnp.zeros((B, E), jnp.float32)
        for _ in range(NSH):
            rows = []
            for bb in range(B):
                sl = emb_ref[bb * T:(bb + 1) * T, :]
                rows.append(jax.lax.dot_general(
                    qt[bb:bb + 1, :], sl, (((1,), (1,)), ((), ())),
                    preferred_element_type=jnp.float32))
            logit = jnp.concatenate(rows, axis=0)
            logit = jnp.where(valid, logit, NEG)
            m = jnp.max(logit, axis=1, keepdims=True)
            w = jnp.exp(logit - m)
            s = jnp.sum(w, axis=1, keepdims=True)
            rows = []
            for bb in range(B):
                sl = emb_ref[bb * T:(bb + 1) * T, :]
                rows.append(jax.lax.dot_general(
                    w[bb:bb + 1, :], sl, (((1,), (0,)), ((), ())),
                    preferred_element_type=jnp.float32))
            attended = jnp.concatenate(rows, axis=0) / s
            gates = (jnp.dot(attended, wih_ref[:], preferred_element_type=jnp.float32)
                     + jnp.dot(qt, whh_ref[:], preferred_element_type=jnp.float32)
                     + bg_ref[:])
            i_g = jax.nn.sigmoid(gates[:, :LSTM])
            f_g = jax.nn.sigmoid(gates[:, LSTM:2 * LSTM])
            g_g = jnp.tanh(gates[:, 2 * LSTM:3 * LSTM])
            o_g = jax.nn.sigmoid(gates[:, 3 * LSTM:])
            ct = f_g * ct + i_g * g_g
            qt = o_g * jnp.tanh(ct)
        out_ref[:, :E] = attended
        out_ref[:, E:] = qt


def _state_imap(b, tb, len_ref):
    last = jax.lax.div(len_ref[b] + T_BLK - 1, T_BLK) - 1
    return (b, jnp.minimum(tb, last), 0)


def _full(b, tb, len_ref):
    return (0, 0)


def kernel(state, length, W1, b1, W2, b2, W_ih, W_hh, b_ih, b_hh):
    length = length.astype(jnp.int32)
    len2d = length.reshape(B, 1)
    grid_spec = pltpu.PrefetchScalarGridSpec(
        num_scalar_prefetch=1,
        grid=(B, TB),
        in_specs=[
            pl.BlockSpec((1, T_BLK, D), _state_imap),
            pl.BlockSpec((B, 1), _full),
            pl.BlockSpec((D, H), _full),
            pl.BlockSpec((1, H), _full),
            pl.BlockSpec((H, E), _full),
            pl.BlockSpec((1, E), _full),
            pl.BlockSpec((E, 4 * LSTM), _full),
            pl.BlockSpec((LSTM, 4 * LSTM), _full),
            pl.BlockSpec((1, 4 * LSTM), _full),
        ],
        out_specs=pl.BlockSpec((B, E + LSTM), _full),
        scratch_shapes=[pltpu.VMEM((B * T, E), jnp.float32)],
    )
    return pl.pallas_call(
        _encoder_kernel,
        grid_spec=grid_spec,
        out_shape=jax.ShapeDtypeStruct((B, E + LSTM), jnp.float32),
        compiler_params=pltpu.CompilerParams(
            dimension_semantics=("arbitrary", "arbitrary")),
    )(length, state, len2d, W1.T.astype(jnp.bfloat16), b1.reshape(1, H),
      W2.T.astype(jnp.bfloat16), b2.reshape(1, E),
      W_ih.T, W_hh.T, (b_ih + b_hh).reshape(1, 4 * LSTM))
